# baseline (speedup 1.0000x reference)
import functools

import jax
import jax.numpy as jnp
from jax import lax
from jax.experimental import pallas as pl
from jax.experimental.pallas import tpu as pltpu
from jax.experimental.pallas import tpu_sc as plsc

_N = 10000
_NPAD = 10016
_D = 128
_E = 320000
_EL = _E + _N
_EPAD = 331776
_ROWS = _EPAD // 128
_NC = 2
_NS = 16
_R1 = _ROWS // _NS
_R2 = _ROWS // (_NC * _NS)
_B = 27
_NB1 = _R1 // _B
_NB2 = _R2 // _B
_L = 16
_ZSPAN = 640


def _tc_prep(x_pad, W, att2):
    def body(x_ref, w_ref, a_ref, h_ref, al_ref):
        h = jnp.dot(x_ref[...], w_ref[...], preferred_element_type=jnp.float32)
        h_ref[...] = h
        al_ref[...] = jnp.dot(h, a_ref[...], preferred_element_type=jnp.float32)

    return pl.pallas_call(
        body,
        out_shape=(
            jax.ShapeDtypeStruct((_NPAD, _D), jnp.float32),
            jax.ShapeDtypeStruct((_NPAD, 2), jnp.float32),
        ),
    )(x_pad, W, att2)


def _tc_finish(part, bias2d):
    def body(p_ref, b_ref, o_ref):
        o_ref[...] = jnp.tanh(p_ref[0, :_N, :] + p_ref[1, :_N, :] + b_ref[...])

    return pl.pallas_call(
        body,
        out_shape=jax.ShapeDtypeStruct((_N, _D), jnp.float32),
    )(part, bias2d)


def _sc_edge(src2d, dst2d, asrc, adst, h):
    mesh = plsc.VectorSubcoreMesh(core_axis_name="c", subcore_axis_name="s")

    @functools.partial(
        pl.kernel,
        out_type=jax.ShapeDtypeStruct((_NC, _NPAD, _D), jnp.float32),
        mesh=mesh,
        compiler_params=pltpu.CompilerParams(needs_layout_passes=False),
        scratch_types=[
            pltpu.VMEM((_B, 128), jnp.int32),
            pltpu.VMEM((_B, 128), jnp.int32),
            pltpu.VMEM((_B, 128), jnp.float32),
            pltpu.VMEM((_NPAD,), jnp.float32),
            pltpu.VMEM((_NPAD,), jnp.float32),
            pltpu.VMEM((_NPAD,), jnp.float32),
            pltpu.VMEM((_L, _D), jnp.float32),
            pltpu.VMEM((_L, _D), jnp.float32),
            pltpu.VMEM((_L, _D), jnp.float32),
            pltpu.VMEM((_L,), jnp.float32),
            pltpu.VMEM((_L,), jnp.int32),
            pltpu.VMEM((_L,), jnp.int32),
            pltpu.VMEM((_L,), jnp.int32),
            pltpu.VMEM((_ZSPAN,), jnp.float32),
            pltpu.SemaphoreType.DMA,
            pltpu.SemaphoreType.DMA,
            pltpu.SemaphoreType.DMA,
            pltpu.SemaphoreType.DMA,
            pltpu.SemaphoreType.DMA,
            pltpu.SemaphoreType.DMA,
            pltpu.SemaphoreType.DMA,
            pltpu.VMEM_SHARED((_NPAD,), jnp.float32),
            pltpu.VMEM_SHARED((_NPAD, _D), jnp.float32),
        ],
    )
    def k(src_hbm, dst_hbm, asrc_hbm, adst_hbm, h_hbm, out_hbm,
          src_v, dst_v, eexp_v, asrc_v, adst_v, denom_v,
          rows0_v, rows1_v, rows2_v, coef_v,
          idx0_v, idx1_v, idx2_v,
          z1_v, gsem0, gsem1, gsem2, ssem0, ssem1, ssem2, dsem,
          denom_sp, out_sp):
        rows = (rows0_v, rows1_v, rows2_v)
        gsems = (gsem0, gsem1, gsem2)
        ssems = (ssem0, ssem1, ssem2)
        idxs = (idx0_v, idx1_v, idx2_v)
        c = lax.axis_index("c")
        s = lax.axis_index("s")

        pltpu.sync_copy(asrc_hbm, asrc_v)
        pltpu.sync_copy(adst_hbm, adst_v)

        zf = jnp.zeros((_L,), jnp.float32)

        @pl.loop(0, _ZSPAN // _L)
        def _(i):
            z1_v[pl.ds(i * _L, _L)] = zf

        @pl.loop(0, _L)
        def _(i):
            for j in range(_D // _L):
                rows0_v[i, pl.ds(j * _L, _L)] = zf

        @pl.when(s < _NS - 1)
        def _():
            pltpu.sync_copy(z1_v, denom_sp.at[pl.ds(s * _ZSPAN, _ZSPAN)])

        @pl.when(s == _NS - 1)
        def _():
            pltpu.sync_copy(z1_v.at[pl.ds(0, 416)],
                            denom_sp.at[pl.ds((_NS - 1) * _ZSPAN, 416)])

        nzi = jnp.where(s == _NS - 1, 416 // _L, _ZSPAN // _L)

        @pl.loop(0, nzi)
        def _(kk):
            pltpu.sync_copy(rows0_v, out_sp.at[pl.ds(s * _ZSPAN + kk * _L, _L)])

        plsc.subcore_barrier()

        def _p1_row(r):
            for kk in range(128 // _L):
                sl = pl.ds(kk * _L, _L)
                s16 = src_v[r, sl]
                d16 = dst_v[r, sl]
                a = (plsc.load_gather(asrc_v, [s16])
                     + plsc.load_gather(adst_v, [d16]))
                e = jnp.maximum(a, 0.2 * a)
                eexp_v[r, sl] = jnp.exp(e)

        @pl.loop(0, _NB1)
        def _(b):
            pltpu.sync_copy(src_hbm.at[s, b], src_v)
            pltpu.sync_copy(dst_hbm.at[s, b], dst_v)

            @pl.loop(0, _B)
            def _(r):
                _p1_row(r)
                pltpu.sync_copy(eexp_v.at[r], denom_sp.at[dst_v.at[r]],
                                add=True)

        plsc.subcore_barrier()
        pltpu.sync_copy(denom_sp, denom_v)

        _NU = 128 // _L

        @pl.loop(0, _NB2)
        def _(b):
            pltpu.sync_copy(src_hbm.at[s, c * _NB2 + b], src_v)
            pltpu.sync_copy(dst_hbm.at[s, c * _NB2 + b], dst_v)

            for p in range(2):
                idxs[p][...] = src_v[0, pl.ds(p * _L, _L)]
                pltpu.async_copy(h_hbm.at[idxs[p]], rows[p], gsems[p])

            @pl.loop(0, _B)
            def _(r):
                for kk in range(_NU):
                    p = kk % 3
                    buf = rows[p]
                    pltpu.make_async_copy(
                        h_hbm.at[idxs[p]], buf, gsems[p]).wait()
                    sl = pl.ds(kk * _L, _L)
                    s16 = src_v[r, sl]
                    d16 = dst_v[r, sl]
                    a = (plsc.load_gather(asrc_v, [s16])
                         + plsc.load_gather(adst_v, [d16]))
                    e = jnp.maximum(a, 0.2 * a)
                    ex = jnp.exp(e)
                    den = plsc.load_gather(denom_v, [d16])
                    coef_v[...] = ex / (den + 1e-16)

                    tcol = (kk + 2) % _NU
                    p2 = tcol % 3
                    if kk + 2 < _NU:
                        idxs[p2][...] = src_v[r, pl.ds(tcol * _L, _L)]
                        pltpu.async_copy(h_hbm.at[idxs[p2]], rows[p2],
                                         gsems[p2])
                    else:
                        @pl.when(r + 1 < _B)
                        def _():
                            idxs[p2][...] = src_v[r + 1,
                                                  pl.ds(tcol * _L, _L)]
                            pltpu.async_copy(h_hbm.at[idxs[p2]], rows[p2],
                                             gsems[p2])

                    @pl.loop(0, _L)
                    def _(i):
                        cc = plsc.load_gather(
                            coef_v, [jnp.full((_L,), i, jnp.int32)])
                        for q in range(_D // _L):
                            sq = pl.ds(q * _L, _L)
                            buf[i, sq] = buf[i, sq] * cc

                    pltpu.sync_copy(buf, out_sp.at[d16], add=True)

        plsc.subcore_barrier()

        @pl.loop(0, nzi)
        def _(kk):
            r0 = s * _ZSPAN + kk * _L
            pltpu.sync_copy(out_sp.at[pl.ds(r0, _L)], rows0_v)
            pltpu.sync_copy(rows0_v, out_hbm.at[c, pl.ds(r0, _L)])

    return k(src2d, dst2d, asrc, adst, h)


def kernel(x, edge_index, W, att_src, att_dst, bias):
    src = edge_index[0].astype(jnp.int32)
    dst = edge_index[1].astype(jnp.int32)
    loops = jnp.arange(_N, dtype=jnp.int32)
    padv = jnp.full((_EPAD - _EL,), _NPAD - 1, jnp.int32)
    src_f = jnp.concatenate([src, loops, padv]).reshape(_NS, _NB1, _B, 128)
    dst_f = jnp.concatenate([dst, loops, padv]).reshape(_NS, _NB1, _B, 128)
    x_pad = jnp.concatenate(
        [x, jnp.zeros((_NPAD - _N, _D), jnp.float32)])
    att2 = jnp.stack([att_src, att_dst], axis=1)
    h, al = _tc_prep(x_pad, W, att2)
    part = _sc_edge(src_f, dst_f, al[:, 0], al[:, 1], h)
    return _tc_finish(part, bias.reshape(1, _D))


# speedup vs baseline: 1.0450x; 1.0450x over previous
import functools

import jax
import jax.numpy as jnp
from jax import lax
from jax.experimental import pallas as pl
from jax.experimental.pallas import tpu as pltpu
from jax.experimental.pallas import tpu_sc as plsc

_N = 10000
_NPAD = 10016
_D = 128
_E = 320000
_EL = _E + _N
_EPAD = 331776
_ROWS = _EPAD // 128
_NC = 2
_NS = 16
_R1 = _ROWS // _NS
_R2 = _ROWS // (_NC * _NS)
_B = 27
_NB1 = _R1 // _B
_NB2 = _R2 // _B
_L = 16
_ZSPAN = 640


def _tc_prep(x_pad, W, att2):
    def body(x_ref, w_ref, a_ref, h_ref, al_ref):
        h = jnp.dot(x_ref[...], w_ref[...], preferred_element_type=jnp.float32)
        h_ref[...] = h
        al_ref[...] = jnp.dot(h, a_ref[...], preferred_element_type=jnp.float32)

    return pl.pallas_call(
        body,
        out_shape=(
            jax.ShapeDtypeStruct((_NPAD, _D), jnp.float32),
            jax.ShapeDtypeStruct((_NPAD, 2), jnp.float32),
        ),
    )(x_pad, W, att2)


def _tc_finish(part, bias2d):
    def body(p_ref, b_ref, o_ref):
        o_ref[...] = jnp.tanh(p_ref[0, :_N, :] + p_ref[1, :_N, :] + b_ref[...])

    return pl.pallas_call(
        body,
        out_shape=jax.ShapeDtypeStruct((_N, _D), jnp.float32),
    )(part, bias2d)


def _sc_edge(src2d, dst2d, asrc, adst, h):
    mesh = plsc.VectorSubcoreMesh(core_axis_name="c", subcore_axis_name="s")

    @functools.partial(
        pl.kernel,
        out_type=jax.ShapeDtypeStruct((_NC, _NPAD, _D), jnp.float32),
        mesh=mesh,
        compiler_params=pltpu.CompilerParams(needs_layout_passes=False),
        scratch_types=[
            pltpu.VMEM((_B, 128), jnp.int32),
            pltpu.VMEM((_B, 128), jnp.int32),
            pltpu.VMEM((_B, 128), jnp.float32),
            pltpu.VMEM((_NPAD,), jnp.float32),
            pltpu.VMEM((_NPAD,), jnp.float32),
            pltpu.VMEM((_NPAD,), jnp.float32),
            pltpu.VMEM((_L, _D), jnp.float32),
            pltpu.VMEM((_L, _D), jnp.float32),
            pltpu.VMEM((_L, _D), jnp.float32),
            pltpu.VMEM((_L,), jnp.float32),
            pltpu.VMEM((_L,), jnp.int32),
            pltpu.VMEM((_L,), jnp.int32),
            pltpu.VMEM((_L,), jnp.int32),
            pltpu.VMEM((_ZSPAN,), jnp.float32),
            pltpu.SemaphoreType.DMA,
            pltpu.SemaphoreType.DMA,
            pltpu.SemaphoreType.DMA,
            pltpu.SemaphoreType.DMA,
            pltpu.SemaphoreType.DMA,
            pltpu.SemaphoreType.DMA,
            pltpu.SemaphoreType.DMA,
            pltpu.VMEM_SHARED((_NPAD,), jnp.float32),
            pltpu.VMEM_SHARED((_NPAD, _D), jnp.float32),
        ],
    )
    def k(src_hbm, dst_hbm, asrc_hbm, adst_hbm, h_hbm, out_hbm,
          src_v, dst_v, eexp_v, asrc_v, adst_v, denom_v,
          rows0_v, rows1_v, rows2_v, coef_v,
          idx0_v, idx1_v, idx2_v,
          z1_v, gsem0, gsem1, gsem2, ssem0, ssem1, ssem2, dsem,
          denom_sp, out_sp):
        rows = (rows0_v, rows1_v, rows2_v)
        gsems = (gsem0, gsem1, gsem2)
        ssems = (ssem0, ssem1, ssem2)
        idxs = (idx0_v, idx1_v, idx2_v)
        c = lax.axis_index("c")
        s = lax.axis_index("s")

        pltpu.sync_copy(asrc_hbm, asrc_v)
        pltpu.sync_copy(adst_hbm, adst_v)

        zf = jnp.zeros((_L,), jnp.float32)

        @pl.loop(0, _ZSPAN // _L)
        def _(i):
            z1_v[pl.ds(i * _L, _L)] = zf

        @pl.loop(0, _L)
        def _(i):
            for j in range(_D // _L):
                rows0_v[i, pl.ds(j * _L, _L)] = zf

        @pl.when(s < _NS - 1)
        def _():
            pltpu.sync_copy(z1_v, denom_sp.at[pl.ds(s * _ZSPAN, _ZSPAN)])

        @pl.when(s == _NS - 1)
        def _():
            pltpu.sync_copy(z1_v.at[pl.ds(0, 416)],
                            denom_sp.at[pl.ds((_NS - 1) * _ZSPAN, 416)])

        nzi = jnp.where(s == _NS - 1, 416 // _L, _ZSPAN // _L)

        @pl.loop(0, nzi)
        def _(kk):
            pltpu.sync_copy(rows0_v, out_sp.at[pl.ds(s * _ZSPAN + kk * _L, _L)])

        plsc.subcore_barrier()

        def _p1_row(r):
            for kk in range(128 // _L):
                sl = pl.ds(kk * _L, _L)
                s16 = src_v[r, sl]
                d16 = dst_v[r, sl]
                a = (plsc.load_gather(asrc_v, [s16])
                     + plsc.load_gather(adst_v, [d16]))
                e = jnp.maximum(a, 0.2 * a)
                eexp_v[r, sl] = jnp.exp(e)

        @pl.loop(0, _NB1)
        def _(b):
            pltpu.sync_copy(src_hbm.at[s, b], src_v)
            pltpu.sync_copy(dst_hbm.at[s, b], dst_v)

            @pl.loop(0, _B)
            def _(r):
                _p1_row(r)
                pltpu.sync_copy(eexp_v.at[r], denom_sp.at[dst_v.at[r]],
                                add=True)

        plsc.subcore_barrier()
        pltpu.sync_copy(denom_sp, denom_v)

        _NU = 128 // _L

        @pl.loop(0, _NB2)
        def _(b):
            pltpu.sync_copy(src_hbm.at[s, c * _NB2 + b], src_v)
            pltpu.sync_copy(dst_hbm.at[s, c * _NB2 + b], dst_v)

            for p in range(2):
                idxs[p][...] = src_v[0, pl.ds(p * _L, _L)]
                pltpu.async_copy(h_hbm.at[idxs[p]], rows[p], gsems[p])

            @pl.loop(0, _B)
            def _(r):
                for kk in range(_NU):
                    p = kk % 3
                    buf = rows[p]
                    sl = pl.ds(kk * _L, _L)
                    s16 = src_v[r, sl]
                    d16 = dst_v[r, sl]
                    a = (plsc.load_gather(asrc_v, [s16])
                         + plsc.load_gather(adst_v, [d16]))
                    e = jnp.maximum(a, 0.2 * a)
                    ex = jnp.exp(e)
                    den = plsc.load_gather(denom_v, [d16])
                    coef_v[...] = ex / (den + 1e-16)
                    pltpu.make_async_copy(
                        h_hbm.at[idxs[p]], buf, gsems[p]).wait()

                    tcol = (kk + 2) % _NU
                    p2 = tcol % 3
                    if kk + 2 < _NU:
                        idxs[p2][...] = src_v[r, pl.ds(tcol * _L, _L)]
                        pltpu.async_copy(h_hbm.at[idxs[p2]], rows[p2],
                                         gsems[p2])
                    else:
                        @pl.when(r + 1 < _B)
                        def _():
                            idxs[p2][...] = src_v[r + 1,
                                                  pl.ds(tcol * _L, _L)]
                            pltpu.async_copy(h_hbm.at[idxs[p2]], rows[p2],
                                             gsems[p2])

                    @pl.loop(0, _L // 2)
                    def _(ih):
                        i0 = ih * 2
                        i1 = ih * 2 + 1
                        cc0 = plsc.load_gather(
                            coef_v, [jnp.full((_L,), i0, jnp.int32)])
                        cc1 = plsc.load_gather(
                            coef_v, [jnp.full((_L,), i1, jnp.int32)])
                        for q in range(_D // _L):
                            sq = pl.ds(q * _L, _L)
                            buf[i0, sq] = buf[i0, sq] * cc0
                            buf[i1, sq] = buf[i1, sq] * cc1

                    pltpu.sync_copy(buf, out_sp.at[d16], add=True)

        plsc.subcore_barrier()

        @pl.when(s < _NS - 1)
        def _():
            pltpu.sync_copy(out_sp.at[pl.ds(s * _ZSPAN, _ZSPAN)],
                            out_hbm.at[c, pl.ds(s * _ZSPAN, _ZSPAN)])

        @pl.when(s == _NS - 1)
        def _():
            pltpu.sync_copy(
                out_sp.at[pl.ds((_NS - 1) * _ZSPAN, 416)],
                out_hbm.at[c, pl.ds((_NS - 1) * _ZSPAN, 416)])

    return k(src2d, dst2d, asrc, adst, h)


def kernel(x, edge_index, W, att_src, att_dst, bias):
    src = edge_index[0].astype(jnp.int32)
    dst = edge_index[1].astype(jnp.int32)
    loops = jnp.arange(_N, dtype=jnp.int32)
    padv = jnp.full((_EPAD - _EL,), _NPAD - 1, jnp.int32)
    src_f = jnp.concatenate([src, loops, padv]).reshape(_NS, _NB1, _B, 128)
    dst_f = jnp.concatenate([dst, loops, padv]).reshape(_NS, _NB1, _B, 128)
    x_pad = jnp.concatenate(
        [x, jnp.zeros((_NPAD - _N, _D), jnp.float32)])
    att2 = jnp.stack([att_src, att_dst], axis=1)
    h, al = _tc_prep(x_pad, W, att2)
    part = _sc_edge(src_f, dst_f, al[:, 0], al[:, 1], h)
    return _tc_finish(part, bias.reshape(1, _D))
